# Initial kernel scaffold; baseline (speedup 1.0000x reference)
import jax
import jax.numpy as jnp
from jax.experimental import pallas as pl


def kernel(user_emb, item_emb, adj_indices, adj_values):
    raise NotImplementedError("write your pallas kernel here")


# speedup vs baseline: 2.9726x; 2.9726x over previous
import jax
import jax.numpy as jnp
from jax import lax
from jax.experimental import pallas as pl
from jax.experimental.pallas import tpu as pltpu
from jax.experimental.pallas import tpu_sc as plsc

N_USER = 25000
N_NODES = 50000
D = 64
H = 32
N_EDGES = 800000
N_LAYERS = 3
NPAD = 50048
K = 128
NSUB = 16
NCHUNK = -(-N_EDGES // (NSUB * K))
EPT = NCHUNK * K
E_PAD = EPT * NSUB
RPT = NPAD // NSUB


def _sc_body(t0, src, dst, vals, zeros, o1, o2, o3,
             acc, srcbuf, dstbuf, valbuf, rows, sem):
    c = lax.axis_index("c")
    s = lax.axis_index("s")
    col_off = c * NPAD
    ebase = s * EPT

    tables = [t0, o1, o2, o3]
    for layer in range(N_LAYERS):
        t_in = tables[layer]
        t_out = tables[layer + 1]

        @pl.when(s == 0)
        def _zero():
            pltpu.sync_copy(zeros, acc)

        plsc.subcore_barrier()

        def chunk_body(i, carry):
            base = ebase + i * K
            pltpu.sync_copy(src.at[pl.ds(base, K)], srcbuf)
            pltpu.sync_copy(dst.at[pl.ds(base, K)], dstbuf)
            pltpu.sync_copy(vals.at[pl.ds(base, K)], valbuf)
            for g in range(K // 16):
                sl = pl.ds(g * 16, 16)
                srcbuf[sl] = srcbuf[sl] + col_off
            pltpu.async_copy(t_in.at[srcbuf], rows, sem).wait()

            def scale(g, carry2):
                vv = valbuf[pl.ds(g * 16, 16)]
                for j in range(16):
                    e = g * 16 + j
                    v = vv[j]
                    rows[e, pl.ds(0, 16)] = rows[e, pl.ds(0, 16)] * v
                    rows[e, pl.ds(16, 16)] = rows[e, pl.ds(16, 16)] * v
                return carry2

            lax.fori_loop(0, K // 16, scale, 0)
            pltpu.sync_copy(rows, acc.at[dstbuf], add=True)
            return carry

        lax.fori_loop(0, NCHUNK, chunk_body, 0)
        plsc.subcore_barrier()

        pltpu.sync_copy(acc.at[pl.ds(s * RPT, RPT)],
                        t_out.at[pl.ds(col_off + s * RPT, RPT)])
        plsc.subcore_barrier()


_sc_propagate = pl.kernel(
    _sc_body,
    out_type=[jax.ShapeDtypeStruct((2 * NPAD, H), jnp.float32)] * N_LAYERS,
    mesh=plsc.VectorSubcoreMesh(core_axis_name="c", subcore_axis_name="s"),
    compiler_params=pltpu.CompilerParams(use_tc_tiling_on_sc=False),
    scratch_types=[
        pltpu.VMEM_SHARED((NPAD, H), jnp.float32),
        pltpu.VMEM((K,), jnp.int32),
        pltpu.VMEM((K,), jnp.int32),
        pltpu.VMEM((K,), jnp.float32),
        pltpu.VMEM((K, H), jnp.float32),
        pltpu.SemaphoreType.DMA,
    ],
)


def kernel(user_emb, item_emb, adj_indices, adj_values):
    all_emb = jnp.concatenate([user_emb, item_emb], axis=0)
    rpad = jnp.zeros((NPAD - N_NODES, H), jnp.float32)
    t0 = jnp.concatenate([all_emb[:, :H], rpad, all_emb[:, H:], rpad], axis=0)

    dst = adj_indices[0]
    src = adj_indices[1]
    pad = E_PAD - N_EDGES
    src_p = jnp.concatenate([src, jnp.zeros((pad,), jnp.int32)])
    dst_p = jnp.concatenate([dst, jnp.zeros((pad,), jnp.int32)])
    val_p = jnp.concatenate([adj_values, jnp.zeros((pad,), jnp.float32)])
    zeros = jnp.zeros((NPAD, H), jnp.float32)

    o1, o2, o3 = _sc_propagate(t0, src_p, dst_p, val_p, zeros)

    def unsplit(t):
        return jnp.concatenate([t[:N_NODES], t[NPAD:NPAD + N_NODES]], axis=1)

    mean = (all_emb + unsplit(o1) + unsplit(o2) + unsplit(o3)) * 0.25
    return mean[:N_USER], mean[N_USER:]
